# baseline (speedup 1.0000x reference)
import functools

import jax
import jax.numpy as jnp
from jax import lax
from jax.experimental import pallas as pl
from jax.experimental.pallas import tpu as pltpu
from jax.experimental.pallas import tpu_sc as plsc

DM = 4096
NF = 32768
NFA = 26624
NFB = NF - NFA
BT = 32
L = 16
NC, NS = 2, 16
NVA = NFA // L
NVB = NFB // L
GROWS = 8
CANDBUF = 512
NCH = 4
SG = 16

BN = 512


def _enc_body(x_ref, bdec_ref, w_ref, benc_ref, o_ref):
    xm = x_ref[...] - bdec_ref[...]
    o_ref[...] = (
        jnp.dot(xm, w_ref[...], preferred_element_type=jnp.float32)
        + benc_ref[...]
    )


def _encode_part(x, W_enc, b_enc, b_dec, start, width, tag):
    nblk = width // BN
    blk0 = start // BN
    return pl.pallas_call(
        _enc_body,
        grid=(nblk,),
        in_specs=[
            pl.BlockSpec((BT, DM), lambda i: (0, 0)),
            pl.BlockSpec((1, DM), lambda i: (0, 0)),
            pl.BlockSpec((DM, BN), lambda i, b=blk0: (0, b + i)),
            pl.BlockSpec((1, BN), lambda i, b=blk0: (0, b + i)),
        ],
        out_specs=pl.BlockSpec((BT, BN), lambda i: (0, i)),
        out_shape=jax.ShapeDtypeStruct((BT, width), jnp.float32),
        name=f"enc{tag}",
    )(x, b_dec.reshape(1, DM), W_enc, b_enc.reshape(1, NF))


def _merge16(hik, hii, lok, loi, sk, si):
    rk = lax.rev(sk, (0,))
    ri = lax.rev(si, (0,))
    p = lok >= rk
    ak = jnp.where(p, lok, rk)
    ai = jnp.where(p, loi, ri)
    ak, ai = plsc.sort_key_val(ak, ai, descending=True)
    rk = lax.rev(ak, (0,))
    ri = lax.rev(ai, (0,))
    p = hik >= rk
    nk = jnp.where(p, hik, rk)
    ni = jnp.where(p, hii, ri)
    mk = jnp.where(p, rk, hik)
    mi = jnp.where(p, ri, hii)
    nk, ni = plsc.sort_key_val(nk, ni, descending=True)
    mk, mi = plsc.sort_key_val(mk, mi, descending=True)
    return nk, ni, mk, mi


_NEG = -3.0e38


def _compact_topk(row_v, nv, cands, candis, t0, idx_base, init_top):
    lane = lax.iota(jnp.int32, L)
    neg = jnp.float32(_NEG)
    t0v = jnp.full((L,), t0)

    def drain(ptrs, top):
        ones = lane >= 0
        for c in range(NCH):
            plsc.store_compressed(cands[c].at[pl.ds(ptrs[c], L)],
                                  jnp.full((L,), neg), mask=ones)

        def dbody(b, c, cv=None, civ=None):
            v = cv[pl.ds(b * L, L)]
            iv = civ[pl.ds(b * L, L)]

            def do(c):
                hik, hii, lok, loi, _ = c
                sk, si = plsc.sort_key_val(v, iv, descending=True)
                hik, hii, lok, loi = _merge16(hik, hii, lok, loi, sk, si)
                thr = jnp.maximum(t0, lax.reduce_min(lok, (0,)))
                return hik, hii, lok, loi, thr

            vmax = lax.reduce_max(v, (0,))
            return lax.cond(vmax >= c[4], do, lambda c: c, c)

        for c in range(NCH):
            nb = (ptrs[c] + L - 1) // L
            top = lax.fori_loop(
                0, nb,
                functools.partial(dbody, cv=cands[c], civ=candis[c]), top)
        return top

    def scan_group(g, carry):
        ptrs, top = list(carry[0]), carry[1:]
        base = g * (SG * L)
        for u in range(SG):
            c = u % NCH
            off = base + u * L
            v = row_v[pl.ds(off, L)]
            m = v >= t0v
            plsc.store_compressed(cands[c].at[pl.ds(ptrs[c], L)], v, mask=m)
            plsc.store_compressed(candis[c].at[pl.ds(ptrs[c], L)],
                                  lane + (off + idx_base), mask=m)
            ptrs[c] = ptrs[c] + plsc.all_reduce_population_count(m)[0]

        pmax = jnp.maximum(jnp.maximum(ptrs[0], ptrs[1]),
                           jnp.maximum(ptrs[2], ptrs[3]))

        def flush(c):
            top = drain(c[0], c[1:])
            return ((jnp.int32(0),) * NCH,) + top

        return lax.cond(pmax >= CANDBUF, flush, lambda c: c,
                        (tuple(ptrs),) + top)

    hik, hii, lok, loi = init_top
    thr0 = jnp.maximum(t0, lax.reduce_min(lok, (0,)))
    init = ((jnp.int32(0),) * NCH, hik, hii, lok, loi, thr0)
    out = lax.fori_loop(0, nv // SG, scan_group, init)
    hik, hii, lok, loi, _ = drain(out[0], out[1:])
    return hik, hii, lok, loi


_CAND_SCRATCH = (
    [pltpu.VMEM((CANDBUF + 6 * L,), jnp.float32)] * NCH
    + [pltpu.VMEM((CANDBUF + 6 * L,), jnp.int32)] * NCH
)


def _sc_scan_body(pre_hbm, vout_hbm, iout_hbm,
                  row_v, stv_v, sti_v,
                  cand0_v, cand1_v, cand2_v, cand3_v,
                  candi0_v, candi1_v, candi2_v, candi3_v):
    wid = lax.axis_index("s") * NC + lax.axis_index("c")
    pltpu.sync_copy(pre_hbm.at[wid], row_v)

    neg = jnp.float32(_NEG)

    def boot(i, c):
        ca, cb = c
        base = i * (8 * L)
        for u in range(0, 8, 2):
            ca = jnp.maximum(ca, row_v[pl.ds(base + u * L, L)])
            cb = jnp.maximum(cb, row_v[pl.ds(base + (u + 1) * L, L)])
        return ca, cb

    ca, cb = lax.fori_loop(0, NVA // 8, boot,
                           (jnp.full((L,), neg), jnp.full((L,), neg)))
    t0 = lax.reduce_min(jnp.minimum(ca, cb), (0,))

    init_top = (jnp.full((L,), neg), jnp.zeros((L,), jnp.int32),
                jnp.full((L,), neg), jnp.zeros((L,), jnp.int32))
    hik, hii, lok, loi = _compact_topk(
        row_v, NVA, [cand0_v, cand1_v, cand2_v, cand3_v],
        [candi0_v, candi1_v, candi2_v, candi3_v], t0, 0, init_top)

    stv_v[pl.ds(0, L)] = hik
    stv_v[pl.ds(L, L)] = lok
    sti_v[pl.ds(0, L)] = hii
    sti_v[pl.ds(L, L)] = loi
    pltpu.sync_copy(stv_v, vout_hbm.at[wid])
    pltpu.sync_copy(sti_v, iout_hbm.at[wid])


def _sc_scan(pre1):
    mesh = plsc.VectorSubcoreMesh(
        core_axis_name="c", subcore_axis_name="s",
        num_cores=NC, num_subcores=NS)
    fn = functools.partial(
        pl.kernel,
        out_type=(jax.ShapeDtypeStruct((BT, 2 * L), jnp.float32),
                  jax.ShapeDtypeStruct((BT, 2 * L), jnp.int32)),
        mesh=mesh,
        scratch_types=[
            pltpu.VMEM((NFA,), jnp.float32),
            pltpu.VMEM((2 * L,), jnp.float32),
            pltpu.VMEM((2 * L,), jnp.int32),
        ] + _CAND_SCRATCH,
        compiler_params=pltpu.CompilerParams(needs_layout_passes=False),
        name="sc_scan",
    )(_sc_scan_body)
    return fn(pre1)


def _sc_final_body(pre_hbm, v1_hbm, i1_hbm, wdec_hbm, bdec_hbm,
                   f_hbm, recon_hbm,
                   row_v, fst_v, bufa_v, bufb_v, acc_v, bdec_v, idx_v,
                   stv_v, sti_v,
                   cand0_v, cand1_v, cand2_v, cand3_v,
                   candi0_v, candi1_v, candi2_v, candi3_v,
                   semg_a, semg_b, semb, semf, semr):
    wid = lax.axis_index("s") * NC + lax.axis_index("c")
    cpb = pltpu.async_copy(bdec_hbm, bdec_v, semb)
    cpr = pltpu.async_copy(pre_hbm.at[wid], row_v, semr)
    pltpu.sync_copy(v1_hbm.at[wid], stv_v)
    pltpu.sync_copy(i1_hbm.at[wid], sti_v)

    hik1 = stv_v[pl.ds(0, L)]
    lok1 = stv_v[pl.ds(L, L)]
    hii1 = sti_v[pl.ds(0, L)]
    loi1 = sti_v[pl.ds(L, L)]

    idx_v[pl.ds(0, L)] = hii1
    idx_v[pl.ds(L, L)] = loi1
    gathers = [
        pltpu.async_copy(
            wdec_hbm.at[idx_v.at[pl.ds(0, GROWS)]], bufa_v, semg_a),
        pltpu.async_copy(
            wdec_hbm.at[idx_v.at[pl.ds(GROWS, GROWS)]], bufb_v, semg_b),
    ]

    zero = jnp.zeros((L,), jnp.float32)

    def zbody(i, _):
        base = i * (8 * L)
        for u in range(8):
            fst_v[pl.ds(base + u * L, L)] = zero
        return 0

    lax.fori_loop(0, NF // (8 * L), zbody, 0)

    hik, hii, lok, loi = hik1, hii1, lok1, loi1
    t0 = lax.reduce_min(lok, (0,))
    cpr.wait()

    hik, hii, lok, loi = _compact_topk(
        row_v, NVB, [cand0_v, cand1_v, cand2_v, cand3_v],
        [candi0_v, candi1_v, candi2_v, candi3_v], t0, NFA,
        (hik, hii, lok, loi))

    plsc.store_scatter(fst_v, [hii], jnp.maximum(hik, 0.0))
    plsc.store_scatter(fst_v, [loi], jnp.maximum(lok, 0.0))
    cpf = pltpu.async_copy(fst_v, f_hbm.at[wid], semf)

    def member(tgt):
        acc = tgt != tgt
        for j in range(L):
            acc = acc | (tgt == jnp.full((L,), hii[j]))
            acc = acc | (tgt == jnp.full((L,), loi[j]))
        return acc

    wv_hi = jnp.where(member(hii1), jnp.maximum(hik1, 0.0), 0.0)
    wv_lo = jnp.where(member(loi1), jnp.maximum(lok1, 0.0), 0.0)
    ws_all = [wv_hi[r] for r in range(L)] + [wv_lo[r] for r in range(L)]
    cpb.wait()

    def fma_chunk(buf, ws, src):
        def jbody(j, _):
            for q in range(4):
                o = j * (4 * L) + q * L
                t = [ws[r] * buf[r, pl.ds(o, L)] for r in range(GROWS)]
                t = [t[2 * i] + t[2 * i + 1] for i in range(GROWS // 2)]
                t = [t[2 * i] + t[2 * i + 1] for i in range(GROWS // 4)]
                acc_v[pl.ds(o, L)] = src[pl.ds(o, L)] + t[0] + t[1]
            return 0

        lax.fori_loop(0, DM // (4 * L), jbody, 0)

    nchunks = (2 * L) // GROWS
    for k in range(nchunks):
        gathers[k].wait()
        buf = bufa_v if k % 2 == 0 else bufb_v
        fma_chunk(buf, ws_all[k * GROWS:(k + 1) * GROWS],
                  bdec_v if k == 0 else acc_v)
        if k + 2 < nchunks:
            gathers.append(pltpu.async_copy(
                wdec_hbm.at[idx_v.at[pl.ds((k + 2) * GROWS, GROWS)]],
                buf, semg_a if k % 2 == 0 else semg_b))

    zi = jnp.zeros((L,), jnp.int32)
    for tname in range(3):
        cand0_v[pl.ds(tname * L, L)] = zero
        candi0_v[pl.ds(tname * L, L)] = zi
    nfa_v = jnp.full((L,), jnp.int32(NFA))
    mh = hii >= nfa_v
    ml = loi >= nfa_v
    plsc.store_compressed(cand0_v.at[pl.ds(0, L)],
                          jnp.maximum(hik, 0.0), mask=mh)
    plsc.store_compressed(candi0_v.at[pl.ds(0, L)], hii, mask=mh)
    n1 = plsc.all_reduce_population_count(mh)[0]
    plsc.store_compressed(cand0_v.at[pl.ds(n1, L)],
                          jnp.maximum(lok, 0.0), mask=ml)
    plsc.store_compressed(candi0_v.at[pl.ds(n1, L)], loi, mask=ml)
    nb = n1 + plsc.all_reduce_population_count(ml)[0]
    wb0 = cand0_v[pl.ds(0, L)]
    wb1 = cand0_v[pl.ds(L, L)]
    ws2 = [wb0[r] for r in range(L)] + [wb1[r] for r in range(L)]

    for t in range(nchunks):
        buf = bufa_v if t % 2 == 0 else bufb_v
        sem = semg_a if t % 2 == 0 else semg_b

        def w2(t=t, buf=buf, sem=sem):
            pltpu.async_copy(
                wdec_hbm.at[candi0_v.at[pl.ds(t * GROWS, GROWS)]],
                buf, sem).wait()
            fma_chunk(buf, ws2[t * GROWS:(t + 1) * GROWS], acc_v)

        if t == 0:
            w2()
        else:
            pl.when(nb > t * GROWS)(w2)

    cpf.wait()
    pltpu.sync_copy(acc_v, recon_hbm.at[wid])


def _sc_final(pre2, v1, i1, W_dec, b_dec):
    mesh = plsc.VectorSubcoreMesh(
        core_axis_name="c", subcore_axis_name="s",
        num_cores=NC, num_subcores=NS)
    fn = functools.partial(
        pl.kernel,
        out_type=(jax.ShapeDtypeStruct((BT, NF), jnp.float32),
                  jax.ShapeDtypeStruct((BT, DM), jnp.float32)),
        mesh=mesh,
        scratch_types=[
            pltpu.VMEM((NFB,), jnp.float32),
            pltpu.VMEM((NF,), jnp.float32),
            pltpu.VMEM((GROWS, DM), jnp.float32),
            pltpu.VMEM((GROWS, DM), jnp.float32),
            pltpu.VMEM((DM,), jnp.float32),
            pltpu.VMEM((DM,), jnp.float32),
            pltpu.VMEM((2 * L,), jnp.int32),
            pltpu.VMEM((2 * L,), jnp.float32),
            pltpu.VMEM((2 * L,), jnp.int32),
        ] + _CAND_SCRATCH + [
            pltpu.SemaphoreType.DMA,
            pltpu.SemaphoreType.DMA,
            pltpu.SemaphoreType.DMA,
            pltpu.SemaphoreType.DMA,
            pltpu.SemaphoreType.DMA,
        ],
        compiler_params=pltpu.CompilerParams(needs_layout_passes=False),
        name="sc_final",
    )(_sc_final_body)
    return fn(pre2, v1, i1, W_dec, b_dec)


def kernel(x, W_enc, b_enc, W_dec, b_dec):
    pre1 = _encode_part(x, W_enc, b_enc, b_dec, 0, NFA, "a")
    v1, i1 = _sc_scan(pre1)
    pre2 = _encode_part(x, W_enc, b_enc, b_dec, NFA, NFB, "b")
    f, recon = _sc_final(pre2, v1, i1, W_dec, b_dec)
    return (recon, f)


# speedup vs baseline: 1.0535x; 1.0535x over previous
import functools

import jax
import jax.numpy as jnp
from jax import lax
from jax.experimental import pallas as pl
from jax.experimental.pallas import tpu as pltpu
from jax.experimental.pallas import tpu_sc as plsc

DM = 4096
NF = 32768
NFA = 26624
NFB = NF - NFA
BT = 32
L = 16
NC, NS = 2, 16
NVA = NFA // L
NVB = NFB // L
GROWS = 8
CANDBUF = 512
NCH = 4
SG = 16

BN = 512


def _enc_body(x_ref, bdec_ref, w_ref, benc_ref, o_ref):
    xm = x_ref[...] - bdec_ref[...]
    o_ref[...] = (
        jnp.dot(xm, w_ref[...], preferred_element_type=jnp.float32)
        + benc_ref[...]
    )


def _encode_part(x, W_enc, b_enc, b_dec, start, width, tag):
    nblk = width // BN
    blk0 = start // BN
    return pl.pallas_call(
        _enc_body,
        grid=(nblk,),
        in_specs=[
            pl.BlockSpec((BT, DM), lambda i: (0, 0)),
            pl.BlockSpec((1, DM), lambda i: (0, 0)),
            pl.BlockSpec((DM, BN), lambda i, b=blk0: (0, b + i)),
            pl.BlockSpec((1, BN), lambda i, b=blk0: (0, b + i)),
        ],
        out_specs=pl.BlockSpec((BT, BN), lambda i: (0, i)),
        out_shape=jax.ShapeDtypeStruct((BT, width), jnp.float32),
        name=f"enc{tag}",
    )(x, b_dec.reshape(1, DM), W_enc, b_enc.reshape(1, NF))


def _merge16(hik, hii, lok, loi, sk, si):
    rk = lax.rev(sk, (0,))
    ri = lax.rev(si, (0,))
    p = lok >= rk
    ak = jnp.where(p, lok, rk)
    ai = jnp.where(p, loi, ri)
    ak, ai = plsc.sort_key_val(ak, ai, descending=True)
    rk = lax.rev(ak, (0,))
    ri = lax.rev(ai, (0,))
    p = hik >= rk
    nk = jnp.where(p, hik, rk)
    ni = jnp.where(p, hii, ri)
    mk = jnp.where(p, rk, hik)
    mi = jnp.where(p, ri, hii)
    nk, ni = plsc.sort_key_val(nk, ni, descending=True)
    mk, mi = plsc.sort_key_val(mk, mi, descending=True)
    return nk, ni, mk, mi


_NEG = -3.0e38


def _compact_topk(row_v, nv, cands, candis, t0, idx_base, init_top):
    lane = lax.iota(jnp.int32, L)
    neg = jnp.float32(_NEG)
    t0v = jnp.full((L,), t0)

    def drain(ptrs, top):
        ones = lane >= 0
        for c in range(NCH):
            plsc.store_compressed(cands[c].at[pl.ds(ptrs[c], L)],
                                  jnp.full((L,), neg), mask=ones)

        def dbody(b, c, cv=None, civ=None):
            v = cv[pl.ds(b * L, L)]
            iv = civ[pl.ds(b * L, L)]

            def do(c):
                hik, hii, lok, loi, _ = c
                sk, si = plsc.sort_key_val(v, iv, descending=True)
                hik, hii, lok, loi = _merge16(hik, hii, lok, loi, sk, si)
                thr = jnp.maximum(t0, lax.reduce_min(lok, (0,)))
                return hik, hii, lok, loi, thr

            vmax = lax.reduce_max(v, (0,))
            return lax.cond(vmax >= c[4], do, lambda c: c, c)

        for c in range(NCH):
            nb = (ptrs[c] + L - 1) // L
            top = lax.fori_loop(
                0, nb,
                functools.partial(dbody, cv=cands[c], civ=candis[c]), top)
        return top

    def scan_group(g, carry):
        ptrs, top = list(carry[0]), carry[1:]
        base = g * (SG * L)
        for u in range(SG):
            c = u % NCH
            off = base + u * L
            v = row_v[pl.ds(off, L)]
            m = v >= t0v
            plsc.store_compressed(cands[c].at[pl.ds(ptrs[c], L)], v, mask=m)
            plsc.store_compressed(candis[c].at[pl.ds(ptrs[c], L)],
                                  lane + (off + idx_base), mask=m)
            ptrs[c] = ptrs[c] + plsc.all_reduce_population_count(m)[0]

        pmax = jnp.maximum(jnp.maximum(ptrs[0], ptrs[1]),
                           jnp.maximum(ptrs[2], ptrs[3]))

        def flush(c):
            top = drain(c[0], c[1:])
            return ((jnp.int32(0),) * NCH,) + top

        return lax.cond(pmax >= CANDBUF, flush, lambda c: c,
                        (tuple(ptrs),) + top)

    hik, hii, lok, loi = init_top
    thr0 = jnp.maximum(t0, lax.reduce_min(lok, (0,)))
    init = ((jnp.int32(0),) * NCH, hik, hii, lok, loi, thr0)
    out = lax.fori_loop(0, nv // SG, scan_group, init)
    hik, hii, lok, loi, _ = drain(out[0], out[1:])
    return hik, hii, lok, loi


_CAND_SCRATCH = (
    [pltpu.VMEM((CANDBUF + 6 * L,), jnp.float32)] * NCH
    + [pltpu.VMEM((CANDBUF + 6 * L,), jnp.int32)] * NCH
)


def _sc_scan_body(pre_hbm, vout_hbm, iout_hbm,
                  row_v, stv_v, sti_v,
                  cand0_v, cand1_v, cand2_v, cand3_v,
                  candi0_v, candi1_v, candi2_v, candi3_v):
    wid = lax.axis_index("s") * NC + lax.axis_index("c")
    pltpu.sync_copy(pre_hbm.at[wid], row_v)

    neg = jnp.float32(_NEG)

    def boot(i, c):
        ca, cb = c
        base = i * (8 * L)
        for u in range(0, 8, 2):
            ca = jnp.maximum(ca, row_v[pl.ds(base + u * L, L)])
            cb = jnp.maximum(cb, row_v[pl.ds(base + (u + 1) * L, L)])
        return ca, cb

    ca, cb = lax.fori_loop(0, NVA // 8, boot,
                           (jnp.full((L,), neg), jnp.full((L,), neg)))
    t0 = lax.reduce_min(jnp.minimum(ca, cb), (0,))

    init_top = (jnp.full((L,), neg), jnp.zeros((L,), jnp.int32),
                jnp.full((L,), neg), jnp.zeros((L,), jnp.int32))
    hik, hii, lok, loi = _compact_topk(
        row_v, NVA, [cand0_v, cand1_v, cand2_v, cand3_v],
        [candi0_v, candi1_v, candi2_v, candi3_v], t0, 0, init_top)

    stv_v[pl.ds(0, L)] = hik
    stv_v[pl.ds(L, L)] = lok
    sti_v[pl.ds(0, L)] = hii
    sti_v[pl.ds(L, L)] = loi
    pltpu.sync_copy(stv_v, vout_hbm.at[wid])
    pltpu.sync_copy(sti_v, iout_hbm.at[wid])


def _sc_scan(pre1):
    mesh = plsc.VectorSubcoreMesh(
        core_axis_name="c", subcore_axis_name="s",
        num_cores=NC, num_subcores=NS)
    fn = functools.partial(
        pl.kernel,
        out_type=(jax.ShapeDtypeStruct((BT, 2 * L), jnp.float32),
                  jax.ShapeDtypeStruct((BT, 2 * L), jnp.int32)),
        mesh=mesh,
        scratch_types=[
            pltpu.VMEM((NFA,), jnp.float32),
            pltpu.VMEM((2 * L,), jnp.float32),
            pltpu.VMEM((2 * L,), jnp.int32),
        ] + _CAND_SCRATCH,
        compiler_params=pltpu.CompilerParams(needs_layout_passes=False),
        name="sc_scan",
    )(_sc_scan_body)
    return fn(pre1)


def _sc_final_body(pre_hbm, v1_hbm, i1_hbm, wdec_hbm, bdec_hbm,
                   f_hbm, recon_hbm,
                   row_v, fst_v, bufa_v, bufb_v, acc_v, bdec_v, idx_v,
                   stv_v, sti_v,
                   cand0_v, cand1_v, cand2_v, cand3_v,
                   candi0_v, candi1_v, candi2_v, candi3_v,
                   semg_a, semg_b, semb, semf, semr):
    wid = lax.axis_index("s") * NC + lax.axis_index("c")
    cpb = pltpu.async_copy(bdec_hbm, bdec_v, semb)
    cpr = pltpu.async_copy(pre_hbm.at[wid], row_v, semr)
    pltpu.sync_copy(v1_hbm.at[wid], stv_v)
    pltpu.sync_copy(i1_hbm.at[wid], sti_v)

    zero = jnp.zeros((L,), jnp.float32)

    def zbody(i, _):
        base = i * (8 * L)
        for u in range(8):
            fst_v[pl.ds(base + u * L, L)] = zero
        return 0

    lax.fori_loop(0, NF // (8 * L), zbody, 0)

    hik = stv_v[pl.ds(0, L)]
    lok = stv_v[pl.ds(L, L)]
    hii = sti_v[pl.ds(0, L)]
    loi = sti_v[pl.ds(L, L)]
    t0 = lax.reduce_min(lok, (0,))
    cpr.wait()

    hik, hii, lok, loi = _compact_topk(
        row_v, NVB, [cand0_v, cand1_v, cand2_v, cand3_v],
        [candi0_v, candi1_v, candi2_v, candi3_v], t0, NFA,
        (hik, hii, lok, loi))

    idx_v[pl.ds(0, L)] = hii
    idx_v[pl.ds(L, L)] = loi
    gathers = [
        pltpu.async_copy(
            wdec_hbm.at[idx_v.at[pl.ds(0, GROWS)]], bufa_v, semg_a)
    ]

    plsc.store_scatter(fst_v, [hii], jnp.maximum(hik, 0.0))
    plsc.store_scatter(fst_v, [loi], jnp.maximum(lok, 0.0))
    cpf = pltpu.async_copy(fst_v, f_hbm.at[wid], semf)

    vh = jnp.maximum(hik, 0.0)
    vl = jnp.maximum(lok, 0.0)
    ws_all = [vh[r] for r in range(L)] + [vl[r] for r in range(L)]
    cpb.wait()

    nchunks = (2 * L) // GROWS
    for k in range(nchunks):
        if k + 1 < nchunks:
            gathers.append(pltpu.async_copy(
                wdec_hbm.at[idx_v.at[pl.ds((k + 1) * GROWS, GROWS)]],
                bufb_v if k % 2 == 0 else bufa_v,
                semg_b if k % 2 == 0 else semg_a))
        gathers[k].wait()
        buf = bufa_v if k % 2 == 0 else bufb_v
        ws = ws_all[k * GROWS:(k + 1) * GROWS]
        src = bdec_v if k == 0 else acc_v

        def jbody(j, _, buf=buf, ws=ws, src=src):
            for q in range(4):
                o = j * (4 * L) + q * L
                t = [ws[r] * buf[r, pl.ds(o, L)] for r in range(GROWS)]
                t = [t[2 * i] + t[2 * i + 1] for i in range(GROWS // 2)]
                t = [t[2 * i] + t[2 * i + 1] for i in range(GROWS // 4)]
                acc_v[pl.ds(o, L)] = src[pl.ds(o, L)] + t[0] + t[1]
            return 0

        lax.fori_loop(0, DM // (4 * L), jbody, 0)

    cpf.wait()
    pltpu.sync_copy(acc_v, recon_hbm.at[wid])


def _sc_final(pre2, v1, i1, W_dec, b_dec):
    mesh = plsc.VectorSubcoreMesh(
        core_axis_name="c", subcore_axis_name="s",
        num_cores=NC, num_subcores=NS)
    fn = functools.partial(
        pl.kernel,
        out_type=(jax.ShapeDtypeStruct((BT, NF), jnp.float32),
                  jax.ShapeDtypeStruct((BT, DM), jnp.float32)),
        mesh=mesh,
        scratch_types=[
            pltpu.VMEM((NFB,), jnp.float32),
            pltpu.VMEM((NF,), jnp.float32),
            pltpu.VMEM((GROWS, DM), jnp.float32),
            pltpu.VMEM((GROWS, DM), jnp.float32),
            pltpu.VMEM((DM,), jnp.float32),
            pltpu.VMEM((DM,), jnp.float32),
            pltpu.VMEM((2 * L,), jnp.int32),
            pltpu.VMEM((2 * L,), jnp.float32),
            pltpu.VMEM((2 * L,), jnp.int32),
        ] + _CAND_SCRATCH + [
            pltpu.SemaphoreType.DMA,
            pltpu.SemaphoreType.DMA,
            pltpu.SemaphoreType.DMA,
            pltpu.SemaphoreType.DMA,
            pltpu.SemaphoreType.DMA,
        ],
        compiler_params=pltpu.CompilerParams(needs_layout_passes=False),
        name="sc_final",
    )(_sc_final_body)
    return fn(pre2, v1, i1, W_dec, b_dec)


def kernel(x, W_enc, b_enc, W_dec, b_dec):
    pre1 = _encode_part(x, W_enc, b_enc, b_dec, 0, NFA, "a")
    v1, i1 = _sc_scan(pre1)
    pre2 = _encode_part(x, W_enc, b_enc, b_dec, NFA, NFB, "b")
    f, recon = _sc_final(pre2, v1, i1, W_dec, b_dec)
    return (recon, f)
